# baseline (speedup 1.0000x reference)
import functools

import jax
import jax.numpy as jnp
from jax import lax
from jax.experimental import pallas as pl
from jax.experimental.pallas import tpu as pltpu
from jax.experimental.pallas import tpu_sc as plsc

_SATELLITE_SPACER_LEN = 17
_NUM_GSPS = 360


def _pad_table_tc(table, width):
    V, E = table.shape

    def body(t_ref, out_ref):
        out_ref[...] = jnp.concatenate(
            [t_ref[...], jnp.zeros((V, width - E), jnp.float32)], axis=1)

    return pl.pallas_call(
        body,
        out_shape=jax.ShapeDtypeStruct((V, width), jnp.float32),
    )(table)


def _sc_embedding_gather(table, idx):
    (BN,) = idx.shape
    V, Ep = table.shape
    info = plsc.get_sparse_core_info()
    num_workers = info.num_cores * info.num_subcores
    per_w = BN // num_workers

    mesh = plsc.VectorSubcoreMesh(core_axis_name="c", subcore_axis_name="s")

    @functools.partial(
        pl.kernel,
        mesh=mesh,
        out_type=jax.ShapeDtypeStruct((BN, Ep), jnp.float32),
        scratch_types=[
            pltpu.VMEM((per_w,), jnp.int32),
            pltpu.VMEM((per_w, Ep), jnp.float32),
            pltpu.SemaphoreType.DMA,
        ],
    )
    def gather_kernel(table_hbm, idx_hbm, out_hbm, idx_v, rows_v, sem):
        wid = lax.axis_index("s") * info.num_cores + lax.axis_index("c")
        base = wid * per_w
        pltpu.sync_copy(idx_hbm.at[pl.ds(base, per_w)], idx_v)
        for j in range(per_w // info.num_lanes):
            sl = pl.ds(j * info.num_lanes, info.num_lanes)
            idx_v[sl] = idx_v[sl] + _NUM_GSPS
        pltpu.async_copy(table_hbm.at[idx_v], rows_v, sem).wait()
        pltpu.sync_copy(rows_v, out_hbm.at[pl.ds(base, per_w)])

    return gather_kernel(table, idx)


def _assemble(tf, tft0, az, el, yf, xf, emb, pv, tmask, e_dim, interpret=False):
    B, T, Ft = tf.shape
    _, N, Fp = yf.shape
    Ep = emb.shape[-1]
    E = e_dim
    D = Ft + Ft + 2 + Fp + Fp + _SATELLITE_SPACER_LEN + 1 + E + 1

    def body(tf_ref, tft0_ref, az_ref, el_ref, y_ref, x_ref, emb_ref,
             pv_ref, m_ref, out_ref):
        t_f = tf_ref[0]
        t0 = tft0_ref[0]
        az_ = az_ref[0]
        el_ = el_ref[0]
        y_ = y_ref[0]
        x_ = x_ref[0]
        e_ = emb_ref[0][:, :E]
        p_ = pv_ref[0] * m_ref[0]
        out = jnp.concatenate([
            jnp.broadcast_to(t_f[:, None, :], (T, N, Ft)),
            jnp.broadcast_to(t0[None, :, :], (T, N, Ft)),
            jnp.broadcast_to(az_[:, None, :], (T, N, 1)),
            jnp.broadcast_to(el_[:, None, :], (T, N, 1)),
            jnp.broadcast_to(y_[None], (T, N, Fp)),
            jnp.broadcast_to(x_[None], (T, N, Fp)),
            jnp.zeros((T, N, _SATELLITE_SPACER_LEN + 1), jnp.float32),
            jnp.broadcast_to(e_[None], (T, N, E)),
            p_[:, :, None],
        ], axis=-1)
        out_ref[0] = out

    return pl.pallas_call(
        body,
        grid=(B,),
        in_specs=[
            pl.BlockSpec((1, T, Ft), lambda b: (b, 0, 0)),
            pl.BlockSpec((1, 1, Ft), lambda b: (b, 0, 0)),
            pl.BlockSpec((1, T, 1), lambda b: (b, 0, 0)),
            pl.BlockSpec((1, T, 1), lambda b: (b, 0, 0)),
            pl.BlockSpec((1, N, Fp), lambda b: (b, 0, 0)),
            pl.BlockSpec((1, N, Fp), lambda b: (b, 0, 0)),
            pl.BlockSpec((1, N, Ep), lambda b: (b, 0, 0)),
            pl.BlockSpec((1, T, N), lambda b: (b, 0, 0)),
            pl.BlockSpec((1, T, 1), lambda b: (0, 0, 0)),
        ],
        out_specs=pl.BlockSpec((1, T, N, D), lambda b: (b, 0, 0, 0)),
        out_shape=jax.ShapeDtypeStruct((B, T, N, D), jnp.float32),
        interpret=interpret,
    )(tf, tft0, az, el, yf, xf, emb, pv, tmask)


def kernel(pv, pv_solar_azimuth, pv_solar_elevation, pv_time_utc_fourier,
           pv_time_utc_fourier_t0, pv_y_osgb_fourier, pv_x_osgb_fourier,
           pv_system_row_number, pv_t0_idx, embedding_table):
    B, T, N = pv.shape
    Ft = pv_time_utc_fourier.shape[-1]
    E = embedding_table.shape[-1]
    idx = pv_system_row_number.astype(jnp.int32).reshape(-1)
    table_p = _pad_table_tc(embedding_table, 128)
    emb = _sc_embedding_gather(table_p, idx).reshape(B, N, -1)
    tmask = (jnp.arange(T) <= pv_t0_idx).astype(pv.dtype).reshape(1, T, 1)
    out = _assemble(
        pv_time_utc_fourier,
        pv_time_utc_fourier_t0.reshape(B, 1, Ft),
        pv_solar_azimuth.reshape(B, T, 1),
        pv_solar_elevation.reshape(B, T, 1),
        pv_y_osgb_fourier,
        pv_x_osgb_fourier,
        emb,
        pv,
        tmask,
        e_dim=E,
    )
    return out.reshape(B * T, N, out.shape[-1])


# speedup vs baseline: 1.0314x; 1.0314x over previous
import functools

import jax
import jax.numpy as jnp
from jax import lax
from jax.experimental import pallas as pl
from jax.experimental.pallas import tpu as pltpu
from jax.experimental.pallas import tpu_sc as plsc

_SATELLITE_SPACER_LEN = 17
_NUM_GSPS = 360


def _pad_table_tc(table, width):
    V, E = table.shape

    def body(t_ref, out_ref):
        out_ref[...] = jnp.concatenate(
            [t_ref[...], jnp.zeros((V, width - E), jnp.float32)], axis=1)

    return pl.pallas_call(
        body,
        out_shape=jax.ShapeDtypeStruct((V, width), jnp.float32),
    )(table)


def _sc_embedding_gather(table, idx):
    B, N = idx.shape
    V, Ep = table.shape
    info = plsc.get_sparse_core_info()
    L = info.num_lanes
    num_workers = info.num_cores * info.num_subcores
    rows_per_w = max(1, B // num_workers)

    mesh = plsc.VectorSubcoreMesh(core_axis_name="c", subcore_axis_name="s")

    @functools.partial(
        pl.kernel,
        mesh=mesh,
        out_type=jax.ShapeDtypeStruct((B, N, Ep), jnp.float32),
        scratch_types=[
            pltpu.VMEM((N,), jnp.int32),
            pltpu.VMEM((N, Ep), jnp.float32),
            pltpu.SemaphoreType.DMA,
        ],
    )
    def gather_kernel(table_hbm, idx_hbm, out_hbm, idx_v, rows_v, sem):
        wid = lax.axis_index("s") * info.num_cores + lax.axis_index("c")

        @pl.when(wid * rows_per_w < B)
        def _():
            for r in range(rows_per_w):
                row = wid * rows_per_w + r
                pltpu.sync_copy(idx_hbm.at[row], idx_v)
                for j in range(N // L):
                    sl = pl.ds(j * L, L)
                    idx_v[sl] = idx_v[sl] + _NUM_GSPS
                pltpu.async_copy(table_hbm.at[idx_v], rows_v, sem).wait()
                pltpu.sync_copy(rows_v, out_hbm.at[row])

    return gather_kernel(table, idx)


def _assemble(t0s, tf, tft0, az, el, yf, xf, emb, pv, e_dim, interpret=False):
    B, T, Ft = tf.shape
    _, N, Fp = yf.shape
    Ep = emb.shape[-1]
    E = e_dim
    D = Ft + Ft + 2 + Fp + Fp + _SATELLITE_SPACER_LEN + 1 + E + 1

    def body(t0_ref, tf_ref, tft0_ref, az_ref, el_ref, y_ref, x_ref,
             emb_ref, pv_ref, out_ref):
        b = pl.program_id(0)
        t_f = tf_ref[0]
        t0 = tft0_ref[b]
        az_ = az_ref[b]
        el_ = el_ref[b]
        y_ = y_ref[0]
        x_ = x_ref[0]
        e_ = emb_ref[0][:, :E]
        t_ids = lax.broadcasted_iota(jnp.int32, (T, N), 0)
        p_ = jnp.where(t_ids <= t0_ref[0], pv_ref[0], 0.0)
        out = jnp.concatenate([
            jnp.broadcast_to(t_f[:, None, :], (T, N, Ft)),
            jnp.broadcast_to(t0[None, None, :], (T, N, Ft)),
            jnp.broadcast_to(az_[:, None, None], (T, N, 1)),
            jnp.broadcast_to(el_[:, None, None], (T, N, 1)),
            jnp.broadcast_to(y_[None], (T, N, Fp)),
            jnp.broadcast_to(x_[None], (T, N, Fp)),
            jnp.zeros((T, N, _SATELLITE_SPACER_LEN + 1), jnp.float32),
            jnp.broadcast_to(e_[None], (T, N, E)),
            p_[:, :, None],
        ], axis=-1)
        out_ref[0] = out

    return pl.pallas_call(
        body,
        grid=(B,),
        in_specs=[
            pl.BlockSpec(memory_space=pltpu.SMEM),
            pl.BlockSpec((1, T, Ft), lambda b: (b, 0, 0)),
            pl.BlockSpec((B, Ft), lambda b: (0, 0)),
            pl.BlockSpec((B, T), lambda b: (0, 0)),
            pl.BlockSpec((B, T), lambda b: (0, 0)),
            pl.BlockSpec((1, N, Fp), lambda b: (b, 0, 0)),
            pl.BlockSpec((1, N, Fp), lambda b: (b, 0, 0)),
            pl.BlockSpec((1, N, Ep), lambda b: (b, 0, 0)),
            pl.BlockSpec((1, T, N), lambda b: (b, 0, 0)),
        ],
        out_specs=pl.BlockSpec((1, T, N, D), lambda b: (b, 0, 0, 0)),
        out_shape=jax.ShapeDtypeStruct((B, T, N, D), jnp.float32),
        interpret=interpret,
    )(t0s, tf, tft0, az, el, yf, xf, emb, pv)


def kernel(pv, pv_solar_azimuth, pv_solar_elevation, pv_time_utc_fourier,
           pv_time_utc_fourier_t0, pv_y_osgb_fourier, pv_x_osgb_fourier,
           pv_system_row_number, pv_t0_idx, embedding_table):
    B, T, N = pv.shape
    E = embedding_table.shape[-1]
    table_p = _pad_table_tc(embedding_table, 128)
    emb = _sc_embedding_gather(table_p,
                               pv_system_row_number.astype(jnp.int32))
    t0s = jnp.asarray(pv_t0_idx, jnp.int32).reshape(1)
    out = _assemble(
        t0s,
        pv_time_utc_fourier,
        pv_time_utc_fourier_t0,
        pv_solar_azimuth,
        pv_solar_elevation,
        pv_y_osgb_fourier,
        pv_x_osgb_fourier,
        emb,
        pv,
        e_dim=E,
    )
    return out.reshape(B * T, N, out.shape[-1])
